# baseline (speedup 1.0000x reference)
import dataclasses
import functools

import jax
import jax.numpy as jnp
from jax import lax
from jax.experimental import pallas as pl
from jax.experimental.pallas import tpu as pltpu
from jax.experimental.pallas import tpu_sc as plsc

N = 10000
D = 128
H = 8
C = 16
HID = 128
NG = 64
E = 320000

NC = 2
NS = 16
NW = NC * NS
EP = 331776
EPW = EP // NW
CHUNK = 64
NCHUNK = EPW // CHUNK
ROW = 144
ACC_R = 10112
ZR = ACC_R // NS

RBLK = 1000
GRID_N = N // RBLK

_EPS_DEN = 1e-16
_EPS_LN = 1e-5


def _compiler_params():
    cp = pltpu.CompilerParams()
    fields = pltpu.CompilerParams.__dataclass_fields__
    if "needs_layout_passes" in fields:
        cp = dataclasses.replace(cp, needs_layout_passes=False)
    if "use_tc_tiling_on_sc" in fields:
        cp = dataclasses.replace(cp, use_tc_tiling_on_sc=False)
    return cp


@functools.lru_cache(maxsize=1)
def _build_sc_edge():
    mesh = plsc.VectorSubcoreMesh(core_axis_name="c", subcore_axis_name="s")
    return pl.kernel(
        _sc_edge_body,
        out_type=jax.ShapeDtypeStruct((NC, ACC_R, ROW), jnp.float32),
        mesh=mesh,
        scratch_types=[
            pltpu.VMEM((CHUNK,), jnp.int32),
            pltpu.VMEM((CHUNK,), jnp.int32),
            pltpu.VMEM((CHUNK, D), jnp.float32),
            pltpu.VMEM((CHUNK, D), jnp.float32),
            pltpu.VMEM((CHUNK, ROW), jnp.float32),
            pltpu.VMEM((H, C), jnp.float32),
            pltpu.VMEM_SHARED((ACC_R, ROW), jnp.float32),
        ],
        compiler_params=_compiler_params(),
    )


def _sc_edge(xl, xrp, src, dst, att, zrow):
    return _build_sc_edge()(xl, xrp, src, dst, att, zrow)


def _sc_edge_body(xl_hbm, xr_hbm, src_hbm, dst_hbm, att_hbm, zero_hbm, out_hbm,
                  idx_s, idx_d, a_buf, b_buf, y_buf, att_v, acc):
    c = lax.axis_index("c")
    s = lax.axis_index("s")
    wid = c * NS + s

    pltpu.sync_copy(att_hbm, att_v)
    pltpu.sync_copy(zero_hbm, acc.at[pl.ds(s * ZR, ZR)])
    plsc.subcore_barrier()

    attv = [att_v[h, :] for h in range(H)]
    lanes = lax.iota(jnp.int32, 16)
    onehot = [(lanes == h).astype(jnp.float32) for h in range(H)]

    @pl.loop(0, NCHUNK)
    def _chunk(k):
        base = wid * EPW + k * CHUNK
        pltpu.sync_copy(src_hbm.at[pl.ds(base, CHUNK)], idx_s)
        pltpu.sync_copy(dst_hbm.at[pl.ds(base, CHUNK)], idx_d)
        pltpu.sync_copy(xl_hbm.at[idx_s], a_buf)
        pltpu.sync_copy(xr_hbm.at[idx_d], b_buf)

        @plsc.parallel_loop(0, CHUNK, unroll=2)
        def _edge(e):
            exs = []
            for h in range(H):
                av = a_buf[e, pl.ds(h * C, C)]
                bv = b_buf[e, pl.ds(h * C, C)]
                z = av + bv
                zl = jnp.maximum(z, 0.2 * z)
                logit = jnp.sum(zl * attv[h])
                exv = jnp.exp(jnp.broadcast_to(logit, (16,)))
                y_buf[e, pl.ds(h * C, C)] = av * exv
                exs.append(exv * onehot[h])
            d0 = (exs[0] + exs[1]) + (exs[2] + exs[3])
            d1 = (exs[4] + exs[5]) + (exs[6] + exs[7])
            y_buf[e, pl.ds(128, 16)] = d0 + d1

        pltpu.sync_copy(y_buf, acc.at[idx_d], add=True)

    plsc.subcore_barrier()
    pltpu.sync_copy(acc.at[pl.ds(s * ZR, ZR)], out_hbm.at[c, pl.ds(s * ZR, ZR)])


def _mm2_body(x_ref, wl_ref, wr_ref, ol_ref, or_ref):
    xb = x_ref[...]
    ol_ref[...] = jnp.dot(xb, wl_ref[...], preferred_element_type=jnp.float32)
    or_ref[...] = jnp.dot(xb, wr_ref[...], preferred_element_type=jnp.float32)


def _mm2(x, wl, wr):
    return pl.pallas_call(
        _mm2_body,
        grid=(GRID_N,),
        in_specs=[
            pl.BlockSpec((RBLK, D), lambda i: (i, 0)),
            pl.BlockSpec((D, HID), lambda i: (0, 0)),
            pl.BlockSpec((D, HID), lambda i: (0, 0)),
        ],
        out_specs=[
            pl.BlockSpec((RBLK, HID), lambda i: (i, 0)),
            pl.BlockSpec((RBLK, HID), lambda i: (i, 0)),
        ],
        out_shape=[jax.ShapeDtypeStruct((N, HID), jnp.float32)] * 2,
    )(x, wl, wr)


def _node_post(acc_ref, b_ref, g_ref, be_ref):
    p = acc_ref[0] + acc_ref[1]
    y = p[:, 0:128]
    den8 = p[:, 128:136]
    hh = lax.broadcasted_iota(jnp.int32, (H, HID), 0)
    cc = lax.broadcasted_iota(jnp.int32, (H, HID), 1) // C
    sel = (hh == cc).astype(jnp.float32)
    den = jnp.dot(den8, sel, preferred_element_type=jnp.float32)
    hcat = y / (den + _EPS_DEN) + b_ref[...]
    mu = jnp.mean(hcat, axis=1, keepdims=True)
    var = jnp.mean((hcat - mu) ** 2, axis=1, keepdims=True)
    hn = g_ref[...] * (hcat - mu) * lax.rsqrt(var + _EPS_LN) + be_ref[...]
    return hn * jax.nn.sigmoid(hn)


def _post_mm_body(acc_ref, b_ref, g_ref, be_ref, wl_ref, wr_ref, ol_ref, or_ref):
    hs = _node_post(acc_ref, b_ref, g_ref, be_ref)
    ol_ref[...] = jnp.dot(hs, wl_ref[...], preferred_element_type=jnp.float32)
    or_ref[...] = jnp.dot(hs, wr_ref[...], preferred_element_type=jnp.float32)


def _post_mm(acc, b, g, be, wl, wr):
    return pl.pallas_call(
        _post_mm_body,
        grid=(GRID_N,),
        in_specs=[
            pl.BlockSpec((NC, RBLK, ROW), lambda i: (0, i, 0)),
            pl.BlockSpec((1, HID), lambda i: (0, 0)),
            pl.BlockSpec((1, HID), lambda i: (0, 0)),
            pl.BlockSpec((1, HID), lambda i: (0, 0)),
            pl.BlockSpec((D, HID), lambda i: (0, 0)),
            pl.BlockSpec((D, HID), lambda i: (0, 0)),
        ],
        out_specs=[
            pl.BlockSpec((RBLK, HID), lambda i: (i, 0)),
            pl.BlockSpec((RBLK, HID), lambda i: (i, 0)),
        ],
        out_shape=[jax.ShapeDtypeStruct((N, HID), jnp.float32)] * 2,
    )(acc, b, g, be, wl, wr)


def _final_body(acc_ref, b_ref, g_ref, be_ref, batch_ref, o_ref, sums_ref, cnt_ref):
    i = pl.program_id(0)
    hs = _node_post(acc_ref, b_ref, g_ref, be_ref)
    bk = batch_ref[0, 0, :]
    onehot = (bk[:, None] == lax.broadcasted_iota(jnp.int32, (RBLK, NG), 1))
    onehot = onehot.astype(jnp.float32)
    dnums = (((0,), (0,)), ((), ()))

    @pl.when(i == 0)
    def _():
        sums_ref[...] = jnp.zeros_like(sums_ref)
        cnt_ref[...] = jnp.zeros_like(cnt_ref)

    sums_ref[...] += lax.dot_general(onehot, hs, dnums,
                                     preferred_element_type=jnp.float32)
    cnt_ref[...] += lax.dot_general(onehot, jnp.ones((RBLK, HID), jnp.float32),
                                    dnums, preferred_element_type=jnp.float32)

    @pl.when(i == GRID_N - 1)
    def _():
        o_ref[...] = sums_ref[...] / jnp.maximum(cnt_ref[...], 1.0)


def _final(acc, b, g, be, batch3):
    return pl.pallas_call(
        _final_body,
        grid=(GRID_N,),
        in_specs=[
            pl.BlockSpec((NC, RBLK, ROW), lambda i: (0, i, 0)),
            pl.BlockSpec((1, HID), lambda i: (0, 0)),
            pl.BlockSpec((1, HID), lambda i: (0, 0)),
            pl.BlockSpec((1, HID), lambda i: (0, 0)),
            pl.BlockSpec((1, 1, RBLK), lambda i: (i, 0, 0)),
        ],
        out_specs=pl.BlockSpec((NG, HID), lambda i: (0, 0)),
        out_shape=jax.ShapeDtypeStruct((NG, HID), jnp.float32),
        scratch_shapes=[
            pltpu.VMEM((NG, HID), jnp.float32),
            pltpu.VMEM((NG, HID), jnp.float32),
        ],
    )(acc, b, g, be, batch3)


def kernel(x, edge_index, batch, Wl0, Wr0, att0, b0, g0, be0,
           Wl1, Wr1, att1, b1, g1, be1):
    loopi = jnp.arange(N, dtype=jnp.int32)
    npad = EP - (E + N)
    src = jnp.concatenate(
        [edge_index[0].astype(jnp.int32), loopi, jnp.zeros((npad,), jnp.int32)])
    dst = jnp.concatenate(
        [edge_index[1].astype(jnp.int32), loopi, jnp.full((npad,), N, jnp.int32)])
    zrow = jnp.zeros((ZR, ROW), jnp.float32)
    zpad = jnp.zeros((1, HID), jnp.float32)
    b0r, g0r, be0r = b0[None, :], g0[None, :], be0[None, :]
    b1r, g1r, be1r = b1[None, :], g1[None, :], be1[None, :]
    batch3 = batch.astype(jnp.int32).reshape(GRID_N, 1, RBLK)

    xl0, xr0 = _mm2(x, Wl0, Wr0)
    xr0p = jnp.concatenate([xr0, zpad], axis=0)
    acc0 = _sc_edge(xl0, xr0p, src, dst, att0, zrow)
    xl1, xr1 = _post_mm(acc0, b0r, g0r, be0r, Wl1, Wr1)
    xr1p = jnp.concatenate([xr1, zpad], axis=0)
    acc1 = _sc_edge(xl1, xr1p, src, dst, att1, zrow)
    return _final(acc1, b1r, g1r, be1r, batch3)


# speedup vs baseline: 1.0244x; 1.0244x over previous
import dataclasses
import functools

import jax
import jax.numpy as jnp
from jax import lax
from jax.experimental import pallas as pl
from jax.experimental.pallas import tpu as pltpu
from jax.experimental.pallas import tpu_sc as plsc

N = 10000
D = 128
H = 8
C = 16
HID = 128
NG = 64
E = 320000

NC = 2
NS = 16
NW = NC * NS
EP = 331776
EPW = EP // NW
CHUNK = 64
NCHUNK = EPW // CHUNK
ROW = 144
ACC_R = 10112
ZR = ACC_R // NS

RBLK = 1000
GRID_N = N // RBLK

_EPS_DEN = 1e-16
_EPS_LN = 1e-5


def _compiler_params():
    cp = pltpu.CompilerParams()
    fields = pltpu.CompilerParams.__dataclass_fields__
    if "needs_layout_passes" in fields:
        cp = dataclasses.replace(cp, needs_layout_passes=False)
    if "use_tc_tiling_on_sc" in fields:
        cp = dataclasses.replace(cp, use_tc_tiling_on_sc=False)
    return cp


@functools.lru_cache(maxsize=1)
def _build_sc_edge():
    mesh = plsc.VectorSubcoreMesh(core_axis_name="c", subcore_axis_name="s")
    return pl.kernel(
        _sc_edge_body,
        out_type=jax.ShapeDtypeStruct((NC, ACC_R, ROW), jnp.float32),
        mesh=mesh,
        scratch_types=[
            pltpu.VMEM((CHUNK,), jnp.int32),
            pltpu.VMEM((CHUNK,), jnp.int32),
            pltpu.VMEM((CHUNK, D), jnp.float32),
            pltpu.VMEM((CHUNK, D), jnp.float32),
            pltpu.VMEM((CHUNK, ROW), jnp.float32),
            pltpu.VMEM((H, C), jnp.float32),
            pltpu.VMEM_SHARED((ACC_R, ROW), jnp.float32),
        ],
        compiler_params=_compiler_params(),
    )


def _sc_edge(xl, xrp, src, dst, att, zrow):
    return _build_sc_edge()(xl, xrp, src, dst, att, zrow)


def _sc_edge_body(xl_hbm, xr_hbm, src_hbm, dst_hbm, att_hbm, zero_hbm, out_hbm,
                  idx_s, idx_d, a_buf, b_buf, y_buf, att_v, acc):
    c = lax.axis_index("c")
    s = lax.axis_index("s")
    wid = c * NS + s

    pltpu.sync_copy(att_hbm, att_v)
    pltpu.sync_copy(zero_hbm, acc.at[pl.ds(s * ZR, ZR)])
    plsc.subcore_barrier()

    attv = [att_v[h, :] for h in range(H)]
    lanes = lax.iota(jnp.int32, 16)
    onehot = [(lanes == h).astype(jnp.float32) for h in range(H)]
    hidx = [jnp.full((16, 1), h, jnp.int32) for h in range(H)]
    dnums = lax.GatherDimensionNumbers(
        offset_dims=(), collapsed_slice_dims=(0,), start_index_map=(0,))

    @pl.loop(0, NCHUNK)
    def _chunk(k):
        base = wid * EPW + k * CHUNK
        pltpu.sync_copy(src_hbm.at[pl.ds(base, CHUNK)], idx_s)
        pltpu.sync_copy(dst_hbm.at[pl.ds(base, CHUNK)], idx_d)
        pltpu.sync_copy(xl_hbm.at[idx_s], a_buf)
        pltpu.sync_copy(xr_hbm.at[idx_d], b_buf)

        @plsc.parallel_loop(0, CHUNK)
        def _edge(e):
            avs, ls = [], []
            for h in range(H):
                av = a_buf[e, pl.ds(h * C, C)]
                bv = b_buf[e, pl.ds(h * C, C)]
                z = av + bv
                zl = jnp.maximum(z, 0.2 * z)
                logit = jnp.sum(zl * attv[h])
                ls.append(jnp.broadcast_to(logit, (16,)) * onehot[h])
                avs.append(av)
            l16 = ((ls[0] + ls[1]) + (ls[2] + ls[3])) + \
                  ((ls[4] + ls[5]) + (ls[6] + ls[7]))
            ex16 = jnp.exp(l16)
            y_buf[e, pl.ds(128, 16)] = ex16
            for h in range(H):
                exv = lax.gather(ex16, hidx[h], dnums, (1,),
                                 mode=lax.GatherScatterMode.PROMISE_IN_BOUNDS)
                y_buf[e, pl.ds(h * C, C)] = avs[h] * exv

        pltpu.sync_copy(y_buf, acc.at[idx_d], add=True)

    plsc.subcore_barrier()
    pltpu.sync_copy(acc.at[pl.ds(s * ZR, ZR)], out_hbm.at[c, pl.ds(s * ZR, ZR)])


def _mm2_body(x_ref, wl_ref, wr_ref, ol_ref, or_ref):
    xb = x_ref[...]
    ol_ref[...] = jnp.dot(xb, wl_ref[...], preferred_element_type=jnp.float32)
    or_ref[...] = jnp.dot(xb, wr_ref[...], preferred_element_type=jnp.float32)


def _mm2(x, wl, wr):
    return pl.pallas_call(
        _mm2_body,
        grid=(GRID_N,),
        in_specs=[
            pl.BlockSpec((RBLK, D), lambda i: (i, 0)),
            pl.BlockSpec((D, HID), lambda i: (0, 0)),
            pl.BlockSpec((D, HID), lambda i: (0, 0)),
        ],
        out_specs=[
            pl.BlockSpec((RBLK, HID), lambda i: (i, 0)),
            pl.BlockSpec((RBLK, HID), lambda i: (i, 0)),
        ],
        out_shape=[jax.ShapeDtypeStruct((N, HID), jnp.float32)] * 2,
    )(x, wl, wr)


def _node_post(acc_ref, b_ref, g_ref, be_ref):
    p = acc_ref[0] + acc_ref[1]
    y = p[:, 0:128]
    den8 = p[:, 128:136]
    hh = lax.broadcasted_iota(jnp.int32, (H, HID), 0)
    cc = lax.broadcasted_iota(jnp.int32, (H, HID), 1) // C
    sel = (hh == cc).astype(jnp.float32)
    den = jnp.dot(den8, sel, preferred_element_type=jnp.float32)
    hcat = y / (den + _EPS_DEN) + b_ref[...]
    mu = jnp.mean(hcat, axis=1, keepdims=True)
    var = jnp.mean((hcat - mu) ** 2, axis=1, keepdims=True)
    hn = g_ref[...] * (hcat - mu) * lax.rsqrt(var + _EPS_LN) + be_ref[...]
    return hn * jax.nn.sigmoid(hn)


def _post_mm_body(acc_ref, b_ref, g_ref, be_ref, wl_ref, wr_ref, ol_ref, or_ref):
    hs = _node_post(acc_ref, b_ref, g_ref, be_ref)
    ol_ref[...] = jnp.dot(hs, wl_ref[...], preferred_element_type=jnp.float32)
    or_ref[...] = jnp.dot(hs, wr_ref[...], preferred_element_type=jnp.float32)


def _post_mm(acc, b, g, be, wl, wr):
    return pl.pallas_call(
        _post_mm_body,
        grid=(GRID_N,),
        in_specs=[
            pl.BlockSpec((NC, RBLK, ROW), lambda i: (0, i, 0)),
            pl.BlockSpec((1, HID), lambda i: (0, 0)),
            pl.BlockSpec((1, HID), lambda i: (0, 0)),
            pl.BlockSpec((1, HID), lambda i: (0, 0)),
            pl.BlockSpec((D, HID), lambda i: (0, 0)),
            pl.BlockSpec((D, HID), lambda i: (0, 0)),
        ],
        out_specs=[
            pl.BlockSpec((RBLK, HID), lambda i: (i, 0)),
            pl.BlockSpec((RBLK, HID), lambda i: (i, 0)),
        ],
        out_shape=[jax.ShapeDtypeStruct((N, HID), jnp.float32)] * 2,
    )(acc, b, g, be, wl, wr)


def _final_body(acc_ref, b_ref, g_ref, be_ref, batch_ref, o_ref, sums_ref, cnt_ref):
    i = pl.program_id(0)
    hs = _node_post(acc_ref, b_ref, g_ref, be_ref)
    bk = batch_ref[0, 0, :]
    onehot = (bk[:, None] == lax.broadcasted_iota(jnp.int32, (RBLK, NG), 1))
    onehot = onehot.astype(jnp.float32)
    dnums = (((0,), (0,)), ((), ()))

    @pl.when(i == 0)
    def _():
        sums_ref[...] = jnp.zeros_like(sums_ref)
        cnt_ref[...] = jnp.zeros_like(cnt_ref)

    sums_ref[...] += lax.dot_general(onehot, hs, dnums,
                                     preferred_element_type=jnp.float32)
    cnt_ref[...] += lax.dot_general(onehot, jnp.ones((RBLK, HID), jnp.float32),
                                    dnums, preferred_element_type=jnp.float32)

    @pl.when(i == GRID_N - 1)
    def _():
        o_ref[...] = sums_ref[...] / jnp.maximum(cnt_ref[...], 1.0)


def _final(acc, b, g, be, batch3):
    return pl.pallas_call(
        _final_body,
        grid=(GRID_N,),
        in_specs=[
            pl.BlockSpec((NC, RBLK, ROW), lambda i: (0, i, 0)),
            pl.BlockSpec((1, HID), lambda i: (0, 0)),
            pl.BlockSpec((1, HID), lambda i: (0, 0)),
            pl.BlockSpec((1, HID), lambda i: (0, 0)),
            pl.BlockSpec((1, 1, RBLK), lambda i: (i, 0, 0)),
        ],
        out_specs=pl.BlockSpec((NG, HID), lambda i: (0, 0)),
        out_shape=jax.ShapeDtypeStruct((NG, HID), jnp.float32),
        scratch_shapes=[
            pltpu.VMEM((NG, HID), jnp.float32),
            pltpu.VMEM((NG, HID), jnp.float32),
        ],
    )(acc, b, g, be, batch3)


def kernel(x, edge_index, batch, Wl0, Wr0, att0, b0, g0, be0,
           Wl1, Wr1, att1, b1, g1, be1):
    loopi = jnp.arange(N, dtype=jnp.int32)
    npad = EP - (E + N)
    src = jnp.concatenate(
        [edge_index[0].astype(jnp.int32), loopi, jnp.zeros((npad,), jnp.int32)])
    dst = jnp.concatenate(
        [edge_index[1].astype(jnp.int32), loopi, jnp.full((npad,), N, jnp.int32)])
    zrow = jnp.zeros((ZR, ROW), jnp.float32)
    zpad = jnp.zeros((1, HID), jnp.float32)
    b0r, g0r, be0r = b0[None, :], g0[None, :], be0[None, :]
    b1r, g1r, be1r = b1[None, :], g1[None, :], be1[None, :]
    batch3 = batch.astype(jnp.int32).reshape(GRID_N, 1, RBLK)

    xl0, xr0 = _mm2(x, Wl0, Wr0)
    xr0p = jnp.concatenate([xr0, zpad], axis=0)
    acc0 = _sc_edge(xl0, xr0p, src, dst, att0, zrow)
    xl1, xr1 = _post_mm(acc0, b0r, g0r, be0r, Wl1, Wr1)
    xr1p = jnp.concatenate([xr1, zpad], axis=0)
    acc1 = _sc_edge(xl1, xr1p, src, dst, att1, zrow)
    return _final(acc1, b1r, g1r, be1r, batch3)


# speedup vs baseline: 1.4842x; 1.4488x over previous
import dataclasses
import functools

import jax
import jax.numpy as jnp
from jax import lax
from jax.experimental import pallas as pl
from jax.experimental.pallas import tpu as pltpu
from jax.experimental.pallas import tpu_sc as plsc

N = 10000
D = 128
H = 8
C = 16
HID = 128
NG = 64
E = 320000

NC = 2
NS = 16
NW = NC * NS
EP = 331776
EPW = EP // NW
CHUNK = 48
NCHUNK = EPW // CHUNK
ROW = 144
ACC_R = 10112
ZR = ACC_R // NS

RBLK = 1000
GRID_N = N // RBLK

_EPS_DEN = 1e-16
_EPS_LN = 1e-5


def _compiler_params():
    cp = pltpu.CompilerParams()
    fields = pltpu.CompilerParams.__dataclass_fields__
    if "needs_layout_passes" in fields:
        cp = dataclasses.replace(cp, needs_layout_passes=False)
    if "use_tc_tiling_on_sc" in fields:
        cp = dataclasses.replace(cp, use_tc_tiling_on_sc=False)
    return cp


@functools.lru_cache(maxsize=1)
def _build_sc_edge():
    mesh = plsc.VectorSubcoreMesh(core_axis_name="c", subcore_axis_name="s")
    return pl.kernel(
        _sc_edge_body,
        out_type=jax.ShapeDtypeStruct((NC, ACC_R, ROW), jnp.float32),
        mesh=mesh,
        scratch_types=[
            pltpu.VMEM((CHUNK,), jnp.int32),
            pltpu.VMEM((CHUNK,), jnp.int32),
            pltpu.VMEM((CHUNK, D), jnp.float32),
            pltpu.VMEM((CHUNK, D), jnp.float32),
            pltpu.VMEM((CHUNK,), jnp.int32),
            pltpu.VMEM((CHUNK,), jnp.int32),
            pltpu.VMEM((CHUNK, D), jnp.float32),
            pltpu.VMEM((CHUNK, D), jnp.float32),
            pltpu.VMEM((CHUNK, ROW), jnp.float32),
            pltpu.VMEM((H, C), jnp.float32),
            pltpu.VMEM_SHARED((ACC_R, ROW), jnp.float32),
            pltpu.SemaphoreType.DMA,
            pltpu.SemaphoreType.DMA,
            pltpu.SemaphoreType.DMA,
            pltpu.SemaphoreType.DMA,
        ],
        compiler_params=_compiler_params(),
    )


def _sc_edge(xl, xrp, src, dst, att, zrow):
    return _build_sc_edge()(xl, xrp, src, dst, att, zrow)


def _sc_edge_body(xl_hbm, xr_hbm, src_hbm, dst_hbm, att_hbm, zero_hbm, out_hbm,
                  idx_s0, idx_d0, a0, b0, idx_s1, idx_d1, a1, b1,
                  y_buf, att_v, acc, sa0, sb0, sa1, sb1):
    c = lax.axis_index("c")
    s = lax.axis_index("s")
    wid = c * NS + s

    pltpu.sync_copy(att_hbm, att_v)
    pltpu.sync_copy(zero_hbm, acc.at[pl.ds(s * ZR, ZR)])
    plsc.subcore_barrier()

    attv = [att_v[h, :] for h in range(H)]
    lanes = lax.iota(jnp.int32, 16)
    onehot = [(lanes == h).astype(jnp.float32) for h in range(H)]
    hidx = [jnp.full((16, 1), h, jnp.int32) for h in range(H)]
    dnums = lax.GatherDimensionNumbers(
        offset_dims=(), collapsed_slice_dims=(0,), start_index_map=(0,))
    sets = ((idx_s0, idx_d0, a0, b0, sa0, sb0),
            (idx_s1, idx_d1, a1, b1, sa1, sb1))

    def _fetch(k, idx_s, idx_d, a_buf, b_buf, sa, sb):
        base = wid * EPW + k * CHUNK
        pltpu.sync_copy(src_hbm.at[pl.ds(base, CHUNK)], idx_s)
        pltpu.sync_copy(dst_hbm.at[pl.ds(base, CHUNK)], idx_d)
        pltpu.async_copy(xl_hbm.at[idx_s], a_buf, sa)
        pltpu.async_copy(xr_hbm.at[idx_d], b_buf, sb)

    def _wait(idx_s, idx_d, a_buf, b_buf, sa, sb):
        pltpu.make_async_copy(xl_hbm.at[idx_s], a_buf, sa).wait()
        pltpu.make_async_copy(xr_hbm.at[idx_d], b_buf, sb).wait()

    def _compute(idx_d, a_buf, b_buf):
        @plsc.parallel_loop(0, CHUNK)
        def _edge(e):
            avs, ls = [], []
            for h in range(H):
                av = a_buf[e, pl.ds(h * C, C)]
                bv = b_buf[e, pl.ds(h * C, C)]
                z = av + bv
                zl = jnp.maximum(z, 0.2 * z)
                logit = jnp.sum(zl * attv[h])
                ls.append(jnp.broadcast_to(logit, (16,)) * onehot[h])
                avs.append(av)
            l16 = ((ls[0] + ls[1]) + (ls[2] + ls[3])) + \
                  ((ls[4] + ls[5]) + (ls[6] + ls[7]))
            ex16 = jnp.exp(l16)
            y_buf[e, pl.ds(128, 16)] = ex16
            for h in range(H):
                exv = lax.gather(ex16, hidx[h], dnums, (1,),
                                 mode=lax.GatherScatterMode.PROMISE_IN_BOUNDS)
                y_buf[e, pl.ds(h * C, C)] = avs[h] * exv

        pltpu.sync_copy(y_buf, acc.at[idx_d], add=True)

    _fetch(0, *sets[0])

    @pl.loop(0, NCHUNK // 2)
    def _pair(p):
        k = 2 * p
        _wait(*sets[0])
        _fetch(k + 1, *sets[1])
        _compute(idx_d0, a0, b0)
        _wait(*sets[1])

        @pl.when(p + 1 < NCHUNK // 2)
        def _():
            _fetch(k + 2, *sets[0])

        _compute(idx_d1, a1, b1)

    plsc.subcore_barrier()
    pltpu.sync_copy(acc.at[pl.ds(s * ZR, ZR)], out_hbm.at[c, pl.ds(s * ZR, ZR)])


def _mm2_body(x_ref, wl_ref, wr_ref, ol_ref, or_ref):
    xb = x_ref[...]
    ol_ref[...] = jnp.dot(xb, wl_ref[...], preferred_element_type=jnp.float32)
    or_ref[...] = jnp.dot(xb, wr_ref[...], preferred_element_type=jnp.float32)


def _mm2(x, wl, wr):
    return pl.pallas_call(
        _mm2_body,
        grid=(GRID_N,),
        in_specs=[
            pl.BlockSpec((RBLK, D), lambda i: (i, 0)),
            pl.BlockSpec((D, HID), lambda i: (0, 0)),
            pl.BlockSpec((D, HID), lambda i: (0, 0)),
        ],
        out_specs=[
            pl.BlockSpec((RBLK, HID), lambda i: (i, 0)),
            pl.BlockSpec((RBLK, HID), lambda i: (i, 0)),
        ],
        out_shape=[jax.ShapeDtypeStruct((N, HID), jnp.float32)] * 2,
    )(x, wl, wr)


def _node_post(acc_ref, b_ref, g_ref, be_ref):
    p = acc_ref[0] + acc_ref[1]
    y = p[:, 0:128]
    den8 = p[:, 128:136]
    hh = lax.broadcasted_iota(jnp.int32, (H, HID), 0)
    cc = lax.broadcasted_iota(jnp.int32, (H, HID), 1) // C
    sel = (hh == cc).astype(jnp.float32)
    den = jnp.dot(den8, sel, preferred_element_type=jnp.float32)
    hcat = y / (den + _EPS_DEN) + b_ref[...]
    mu = jnp.mean(hcat, axis=1, keepdims=True)
    var = jnp.mean((hcat - mu) ** 2, axis=1, keepdims=True)
    hn = g_ref[...] * (hcat - mu) * lax.rsqrt(var + _EPS_LN) + be_ref[...]
    return hn * jax.nn.sigmoid(hn)


def _post_mm_body(acc_ref, b_ref, g_ref, be_ref, wl_ref, wr_ref, ol_ref, or_ref):
    hs = _node_post(acc_ref, b_ref, g_ref, be_ref)
    ol_ref[...] = jnp.dot(hs, wl_ref[...], preferred_element_type=jnp.float32)
    or_ref[...] = jnp.dot(hs, wr_ref[...], preferred_element_type=jnp.float32)


def _post_mm(acc, b, g, be, wl, wr):
    return pl.pallas_call(
        _post_mm_body,
        grid=(GRID_N,),
        in_specs=[
            pl.BlockSpec((NC, RBLK, ROW), lambda i: (0, i, 0)),
            pl.BlockSpec((1, HID), lambda i: (0, 0)),
            pl.BlockSpec((1, HID), lambda i: (0, 0)),
            pl.BlockSpec((1, HID), lambda i: (0, 0)),
            pl.BlockSpec((D, HID), lambda i: (0, 0)),
            pl.BlockSpec((D, HID), lambda i: (0, 0)),
        ],
        out_specs=[
            pl.BlockSpec((RBLK, HID), lambda i: (i, 0)),
            pl.BlockSpec((RBLK, HID), lambda i: (i, 0)),
        ],
        out_shape=[jax.ShapeDtypeStruct((N, HID), jnp.float32)] * 2,
    )(acc, b, g, be, wl, wr)


def _final_body(acc_ref, b_ref, g_ref, be_ref, batch_ref, o_ref, sums_ref, cnt_ref):
    i = pl.program_id(0)
    hs = _node_post(acc_ref, b_ref, g_ref, be_ref)
    bk = batch_ref[0, 0, :]
    onehot = (bk[:, None] == lax.broadcasted_iota(jnp.int32, (RBLK, NG), 1))
    onehot = onehot.astype(jnp.float32)
    dnums = (((0,), (0,)), ((), ()))

    @pl.when(i == 0)
    def _():
        sums_ref[...] = jnp.zeros_like(sums_ref)
        cnt_ref[...] = jnp.zeros_like(cnt_ref)

    sums_ref[...] += lax.dot_general(onehot, hs, dnums,
                                     preferred_element_type=jnp.float32)
    cnt_ref[...] += lax.dot_general(onehot, jnp.ones((RBLK, HID), jnp.float32),
                                    dnums, preferred_element_type=jnp.float32)

    @pl.when(i == GRID_N - 1)
    def _():
        o_ref[...] = sums_ref[...] / jnp.maximum(cnt_ref[...], 1.0)


def _final(acc, b, g, be, batch3):
    return pl.pallas_call(
        _final_body,
        grid=(GRID_N,),
        in_specs=[
            pl.BlockSpec((NC, RBLK, ROW), lambda i: (0, i, 0)),
            pl.BlockSpec((1, HID), lambda i: (0, 0)),
            pl.BlockSpec((1, HID), lambda i: (0, 0)),
            pl.BlockSpec((1, HID), lambda i: (0, 0)),
            pl.BlockSpec((1, 1, RBLK), lambda i: (i, 0, 0)),
        ],
        out_specs=pl.BlockSpec((NG, HID), lambda i: (0, 0)),
        out_shape=jax.ShapeDtypeStruct((NG, HID), jnp.float32),
        scratch_shapes=[
            pltpu.VMEM((NG, HID), jnp.float32),
            pltpu.VMEM((NG, HID), jnp.float32),
        ],
    )(acc, b, g, be, batch3)


def kernel(x, edge_index, batch, Wl0, Wr0, att0, b0, g0, be0,
           Wl1, Wr1, att1, b1, g1, be1):
    loopi = jnp.arange(N, dtype=jnp.int32)
    npad = EP - (E + N)
    src = jnp.concatenate(
        [edge_index[0].astype(jnp.int32), loopi, jnp.zeros((npad,), jnp.int32)])
    dst = jnp.concatenate(
        [edge_index[1].astype(jnp.int32), loopi, jnp.full((npad,), N, jnp.int32)])
    zrow = jnp.zeros((ZR, ROW), jnp.float32)
    zpad = jnp.zeros((1, HID), jnp.float32)
    b0r, g0r, be0r = b0[None, :], g0[None, :], be0[None, :]
    b1r, g1r, be1r = b1[None, :], g1[None, :], be1[None, :]
    batch3 = batch.astype(jnp.int32).reshape(GRID_N, 1, RBLK)

    xl0, xr0 = _mm2(x, Wl0, Wr0)
    xr0p = jnp.concatenate([xr0, zpad], axis=0)
    acc0 = _sc_edge(xl0, xr0p, src, dst, att0, zrow)
    xl1, xr1 = _post_mm(acc0, b0r, g0r, be0r, Wl1, Wr1)
    xr1p = jnp.concatenate([xr1, zpad], axis=0)
    acc1 = _sc_edge(xl1, xr1p, src, dst, att1, zrow)
    return _final(acc1, b1r, g1r, be1r, batch3)


# speedup vs baseline: 1.6132x; 1.0869x over previous
import dataclasses
import functools

import jax
import jax.numpy as jnp
from jax import lax
from jax.experimental import pallas as pl
from jax.experimental.pallas import tpu as pltpu
from jax.experimental.pallas import tpu_sc as plsc

N = 10000
D = 128
H = 8
C = 16
HID = 128
NG = 64
E = 320000

NC = 2
NS = 16
NW = NC * NS
EP = 331776
EPW = EP // NW
CHUNK = 48
NCHUNK = EPW // CHUNK
ROW = 144
ACC_R = 10112
ZR = ACC_R // NS

RBLK = 1000
GRID_N = N // RBLK

_EPS_DEN = 1e-16
_EPS_LN = 1e-5


def _compiler_params():
    cp = pltpu.CompilerParams()
    fields = pltpu.CompilerParams.__dataclass_fields__
    if "needs_layout_passes" in fields:
        cp = dataclasses.replace(cp, needs_layout_passes=False)
    if "use_tc_tiling_on_sc" in fields:
        cp = dataclasses.replace(cp, use_tc_tiling_on_sc=False)
    return cp


@functools.lru_cache(maxsize=1)
def _build_sc_edge():
    mesh = plsc.VectorSubcoreMesh(core_axis_name="c", subcore_axis_name="s")
    return pl.kernel(
        _sc_edge_body,
        out_type=jax.ShapeDtypeStruct((NC, ACC_R, ROW), jnp.float32),
        mesh=mesh,
        scratch_types=[
            pltpu.VMEM((CHUNK,), jnp.int32),
            pltpu.VMEM((CHUNK,), jnp.int32),
            pltpu.VMEM((CHUNK, D), jnp.float32),
            pltpu.VMEM((CHUNK, D), jnp.float32),
            pltpu.VMEM((CHUNK,), jnp.int32),
            pltpu.VMEM((CHUNK,), jnp.int32),
            pltpu.VMEM((CHUNK, D), jnp.float32),
            pltpu.VMEM((CHUNK, D), jnp.float32),
            pltpu.VMEM((CHUNK, ROW), jnp.float32),
            pltpu.VMEM((CHUNK, ROW), jnp.float32),
            pltpu.VMEM((CHUNK,), jnp.int32),
            pltpu.VMEM((CHUNK,), jnp.int32),
            pltpu.VMEM((H, C), jnp.float32),
            pltpu.VMEM_SHARED((ACC_R, ROW), jnp.float32),
            pltpu.SemaphoreType.DMA,
            pltpu.SemaphoreType.DMA,
            pltpu.SemaphoreType.DMA,
            pltpu.SemaphoreType.DMA,
            pltpu.SemaphoreType.DMA,
            pltpu.SemaphoreType.DMA,
        ],
        compiler_params=_compiler_params(),
    )


def _sc_edge(xl, xrp, src, dst, att, zrow):
    return _build_sc_edge()(xl, xrp, src, dst, att, zrow)


def _sc_edge_body(xl_hbm, xr_hbm, src_hbm, dst_hbm, att_hbm, zero_hbm, out_hbm,
                  idx_s0, idx_d0, a0, b0, idx_s1, idx_d1, a1, b1,
                  y0, y1, sd0, sd1, att_v, acc,
                  sa0, sb0, sa1, sb1, so0, so1):
    c = lax.axis_index("c")
    s = lax.axis_index("s")
    wid = c * NS + s

    pltpu.sync_copy(att_hbm, att_v)
    pltpu.sync_copy(zero_hbm, acc.at[pl.ds(s * ZR, ZR)])
    plsc.subcore_barrier()

    attv = [att_v[h, :] for h in range(H)]
    lanes = lax.iota(jnp.int32, 16)
    onehot = [(lanes == h).astype(jnp.float32) for h in range(H)]
    hidx = [jnp.full((16, 1), h, jnp.int32) for h in range(H)]
    dnums = lax.GatherDimensionNumbers(
        offset_dims=(), collapsed_slice_dims=(0,), start_index_map=(0,))
    sets = ((idx_s0, idx_d0, a0, b0, sa0, sb0),
            (idx_s1, idx_d1, a1, b1, sa1, sb1))

    def _fetch(k, idx_s, idx_d, a_buf, b_buf, sa, sb):
        base = wid * EPW + k * CHUNK
        pltpu.sync_copy(src_hbm.at[pl.ds(base, CHUNK)], idx_s)
        pltpu.sync_copy(dst_hbm.at[pl.ds(base, CHUNK)], idx_d)
        pltpu.async_copy(xl_hbm.at[idx_s], a_buf, sa)
        pltpu.async_copy(xr_hbm.at[idx_d], b_buf, sb)

    def _wait(idx_s, idx_d, a_buf, b_buf, sa, sb):
        pltpu.make_async_copy(xl_hbm.at[idx_s], a_buf, sa).wait()
        pltpu.make_async_copy(xr_hbm.at[idx_d], b_buf, sb).wait()

    def _compute(idx_d, a_buf, b_buf, y_buf, sd, so, p):
        @pl.when(p > 0)
        def _():
            pltpu.make_async_copy(y_buf, acc.at[sd], so).wait()

        @plsc.parallel_loop(0, CHUNK)
        def _edge(e):
            avs, ls = [], []
            for h in range(H):
                av = a_buf[e, pl.ds(h * C, C)]
                bv = b_buf[e, pl.ds(h * C, C)]
                z = av + bv
                zl = jnp.maximum(z, 0.2 * z)
                logit = jnp.sum(zl * attv[h])
                ls.append(jnp.broadcast_to(logit, (16,)) * onehot[h])
                avs.append(av)
            l16 = ((ls[0] + ls[1]) + (ls[2] + ls[3])) + \
                  ((ls[4] + ls[5]) + (ls[6] + ls[7]))
            ex16 = jnp.exp(l16)
            y_buf[e, pl.ds(128, 16)] = ex16
            for h in range(H):
                exv = lax.gather(ex16, hidx[h], dnums, (1,),
                                 mode=lax.GatherScatterMode.PROMISE_IN_BOUNDS)
                y_buf[e, pl.ds(h * C, C)] = avs[h] * exv

        for i in range(CHUNK // 16):
            sd[pl.ds(i * 16, 16)] = idx_d[pl.ds(i * 16, 16)]
        pltpu.async_copy(y_buf, acc.at[sd], so, add=True)

    _fetch(0, *sets[0])

    @pl.loop(0, NCHUNK // 2)
    def _pair(p):
        k = 2 * p
        _wait(*sets[0])
        _fetch(k + 1, *sets[1])
        _compute(idx_d0, a0, b0, y0, sd0, so0, p)
        _wait(*sets[1])

        @pl.when(p + 1 < NCHUNK // 2)
        def _():
            _fetch(k + 2, *sets[0])

        _compute(idx_d1, a1, b1, y1, sd1, so1, p)

    pltpu.make_async_copy(y0, acc.at[sd0], so0).wait()
    pltpu.make_async_copy(y1, acc.at[sd1], so1).wait()
    plsc.subcore_barrier()
    pltpu.sync_copy(acc.at[pl.ds(s * ZR, ZR)], out_hbm.at[c, pl.ds(s * ZR, ZR)])


def _mm2_body(x_ref, wl_ref, wr_ref, ol_ref, or_ref):
    xb = x_ref[...]
    ol_ref[...] = jnp.dot(xb, wl_ref[...], preferred_element_type=jnp.float32)
    or_ref[...] = jnp.dot(xb, wr_ref[...], preferred_element_type=jnp.float32)


def _mm2(x, wl, wr):
    return pl.pallas_call(
        _mm2_body,
        grid=(GRID_N,),
        in_specs=[
            pl.BlockSpec((RBLK, D), lambda i: (i, 0)),
            pl.BlockSpec((D, HID), lambda i: (0, 0)),
            pl.BlockSpec((D, HID), lambda i: (0, 0)),
        ],
        out_specs=[
            pl.BlockSpec((RBLK, HID), lambda i: (i, 0)),
            pl.BlockSpec((RBLK, HID), lambda i: (i, 0)),
        ],
        out_shape=[jax.ShapeDtypeStruct((N, HID), jnp.float32)] * 2,
    )(x, wl, wr)


def _node_post(acc_ref, b_ref, g_ref, be_ref):
    p = acc_ref[0] + acc_ref[1]
    y = p[:, 0:128]
    den8 = p[:, 128:136]
    hh = lax.broadcasted_iota(jnp.int32, (H, HID), 0)
    cc = lax.broadcasted_iota(jnp.int32, (H, HID), 1) // C
    sel = (hh == cc).astype(jnp.float32)
    den = jnp.dot(den8, sel, preferred_element_type=jnp.float32)
    hcat = y / (den + _EPS_DEN) + b_ref[...]
    mu = jnp.mean(hcat, axis=1, keepdims=True)
    var = jnp.mean((hcat - mu) ** 2, axis=1, keepdims=True)
    hn = g_ref[...] * (hcat - mu) * lax.rsqrt(var + _EPS_LN) + be_ref[...]
    return hn * jax.nn.sigmoid(hn)


def _post_mm_body(acc_ref, b_ref, g_ref, be_ref, wl_ref, wr_ref, ol_ref, or_ref):
    hs = _node_post(acc_ref, b_ref, g_ref, be_ref)
    ol_ref[...] = jnp.dot(hs, wl_ref[...], preferred_element_type=jnp.float32)
    or_ref[...] = jnp.dot(hs, wr_ref[...], preferred_element_type=jnp.float32)


def _post_mm(acc, b, g, be, wl, wr):
    return pl.pallas_call(
        _post_mm_body,
        grid=(GRID_N,),
        in_specs=[
            pl.BlockSpec((NC, RBLK, ROW), lambda i: (0, i, 0)),
            pl.BlockSpec((1, HID), lambda i: (0, 0)),
            pl.BlockSpec((1, HID), lambda i: (0, 0)),
            pl.BlockSpec((1, HID), lambda i: (0, 0)),
            pl.BlockSpec((D, HID), lambda i: (0, 0)),
            pl.BlockSpec((D, HID), lambda i: (0, 0)),
        ],
        out_specs=[
            pl.BlockSpec((RBLK, HID), lambda i: (i, 0)),
            pl.BlockSpec((RBLK, HID), lambda i: (i, 0)),
        ],
        out_shape=[jax.ShapeDtypeStruct((N, HID), jnp.float32)] * 2,
    )(acc, b, g, be, wl, wr)


def _final_body(acc_ref, b_ref, g_ref, be_ref, batch_ref, o_ref, sums_ref, cnt_ref):
    i = pl.program_id(0)
    hs = _node_post(acc_ref, b_ref, g_ref, be_ref)
    bk = batch_ref[0, 0, :]
    onehot = (bk[:, None] == lax.broadcasted_iota(jnp.int32, (RBLK, NG), 1))
    onehot = onehot.astype(jnp.float32)
    dnums = (((0,), (0,)), ((), ()))

    @pl.when(i == 0)
    def _():
        sums_ref[...] = jnp.zeros_like(sums_ref)
        cnt_ref[...] = jnp.zeros_like(cnt_ref)

    sums_ref[...] += lax.dot_general(onehot, hs, dnums,
                                     preferred_element_type=jnp.float32)
    cnt_ref[...] += lax.dot_general(onehot, jnp.ones((RBLK, HID), jnp.float32),
                                    dnums, preferred_element_type=jnp.float32)

    @pl.when(i == GRID_N - 1)
    def _():
        o_ref[...] = sums_ref[...] / jnp.maximum(cnt_ref[...], 1.0)


def _final(acc, b, g, be, batch3):
    return pl.pallas_call(
        _final_body,
        grid=(GRID_N,),
        in_specs=[
            pl.BlockSpec((NC, RBLK, ROW), lambda i: (0, i, 0)),
            pl.BlockSpec((1, HID), lambda i: (0, 0)),
            pl.BlockSpec((1, HID), lambda i: (0, 0)),
            pl.BlockSpec((1, HID), lambda i: (0, 0)),
            pl.BlockSpec((1, 1, RBLK), lambda i: (i, 0, 0)),
        ],
        out_specs=pl.BlockSpec((NG, HID), lambda i: (0, 0)),
        out_shape=jax.ShapeDtypeStruct((NG, HID), jnp.float32),
        scratch_shapes=[
            pltpu.VMEM((NG, HID), jnp.float32),
            pltpu.VMEM((NG, HID), jnp.float32),
        ],
    )(acc, b, g, be, batch3)


def kernel(x, edge_index, batch, Wl0, Wr0, att0, b0, g0, be0,
           Wl1, Wr1, att1, b1, g1, be1):
    loopi = jnp.arange(N, dtype=jnp.int32)
    npad = EP - (E + N)
    src = jnp.concatenate(
        [edge_index[0].astype(jnp.int32), loopi, jnp.zeros((npad,), jnp.int32)])
    dst = jnp.concatenate(
        [edge_index[1].astype(jnp.int32), loopi, jnp.full((npad,), N, jnp.int32)])
    zrow = jnp.zeros((ZR, ROW), jnp.float32)
    zpad = jnp.zeros((1, HID), jnp.float32)
    b0r, g0r, be0r = b0[None, :], g0[None, :], be0[None, :]
    b1r, g1r, be1r = b1[None, :], g1[None, :], be1[None, :]
    batch3 = batch.astype(jnp.int32).reshape(GRID_N, 1, RBLK)

    xl0, xr0 = _mm2(x, Wl0, Wr0)
    xr0p = jnp.concatenate([xr0, zpad], axis=0)
    acc0 = _sc_edge(xl0, xr0p, src, dst, att0, zrow)
    xl1, xr1 = _post_mm(acc0, b0r, g0r, be0r, Wl1, Wr1)
    xr1p = jnp.concatenate([xr1, zpad], axis=0)
    acc1 = _sc_edge(xl1, xr1p, src, dst, att1, zrow)
    return _final(acc1, b1r, g1r, be1r, batch3)


# speedup vs baseline: 1.7506x; 1.0852x over previous
import dataclasses
import functools

import jax
import jax.numpy as jnp
from jax import lax
from jax.experimental import pallas as pl
from jax.experimental.pallas import tpu as pltpu
from jax.experimental.pallas import tpu_sc as plsc

N = 10000
D = 128
H = 8
C = 16
HID = 128
NG = 64
E = 320000

NC = 2
NS = 16
NW = NC * NS
EP = 331776
EPW = EP // NW
CHUNK = 48
NCHUNK = EPW // CHUNK
ROW = 144
ACC_R = 10112
ZR = ACC_R // NS

RBLK = 1000
GRID_N = N // RBLK

_EPS_DEN = 1e-16
_EPS_LN = 1e-5


def _compiler_params():
    cp = pltpu.CompilerParams()
    fields = pltpu.CompilerParams.__dataclass_fields__
    if "needs_layout_passes" in fields:
        cp = dataclasses.replace(cp, needs_layout_passes=False)
    if "use_tc_tiling_on_sc" in fields:
        cp = dataclasses.replace(cp, use_tc_tiling_on_sc=False)
    return cp


@functools.lru_cache(maxsize=1)
def _build_sc_edge():
    mesh = plsc.VectorSubcoreMesh(core_axis_name="c", subcore_axis_name="s")
    return pl.kernel(
        _sc_edge_body,
        out_type=jax.ShapeDtypeStruct((NC, ACC_R, ROW), jnp.float32),
        mesh=mesh,
        scratch_types=[
            pltpu.VMEM((CHUNK,), jnp.int32),
            pltpu.VMEM((CHUNK,), jnp.int32),
            pltpu.VMEM((CHUNK, D), jnp.float32),
            pltpu.VMEM((CHUNK, D), jnp.float32),
            pltpu.VMEM((CHUNK,), jnp.int32),
            pltpu.VMEM((CHUNK,), jnp.int32),
            pltpu.VMEM((CHUNK, D), jnp.float32),
            pltpu.VMEM((CHUNK, D), jnp.float32),
            pltpu.VMEM((CHUNK, ROW), jnp.float32),
            pltpu.VMEM((CHUNK, ROW), jnp.float32),
            pltpu.VMEM((CHUNK,), jnp.int32),
            pltpu.VMEM((CHUNK,), jnp.int32),
            pltpu.VMEM((H, C), jnp.float32),
            pltpu.VMEM_SHARED((ACC_R, ROW), jnp.float32),
            pltpu.SemaphoreType.DMA,
            pltpu.SemaphoreType.DMA,
            pltpu.SemaphoreType.DMA,
            pltpu.SemaphoreType.DMA,
            pltpu.SemaphoreType.DMA,
            pltpu.SemaphoreType.DMA,
        ],
        compiler_params=_compiler_params(),
    )


def _sc_edge(xl, xrp, src, dst, att, zrow):
    return _build_sc_edge()(xl, xrp, src, dst, att, zrow)


def _sc_edge_body(xl_hbm, xr_hbm, src_hbm, dst_hbm, att_hbm, zero_hbm, out_hbm,
                  idx_s0, idx_d0, a0, b0, idx_s1, idx_d1, a1, b1,
                  y0, y1, sd0, sd1, att_v, acc,
                  sa0, sb0, sa1, sb1, so0, so1):
    c = lax.axis_index("c")
    s = lax.axis_index("s")
    wid = c * NS + s

    pltpu.sync_copy(att_hbm, att_v)
    pltpu.sync_copy(zero_hbm, acc.at[pl.ds(s * ZR, ZR)])
    plsc.subcore_barrier()

    attv = [att_v[h, :] for h in range(H)]
    lanes = lax.iota(jnp.int32, 16)
    laneh = [lanes == h for h in range(H)]
    sets = ((idx_s0, idx_d0, a0, b0, sa0, sb0),
            (idx_s1, idx_d1, a1, b1, sa1, sb1))

    def _fetch(k, idx_s, idx_d, a_buf, b_buf, sa, sb):
        base = wid * EPW + k * CHUNK
        pltpu.sync_copy(src_hbm.at[pl.ds(base, CHUNK)], idx_s)
        pltpu.sync_copy(dst_hbm.at[pl.ds(base, CHUNK)], idx_d)
        pltpu.async_copy(xl_hbm.at[idx_s], a_buf, sa)
        pltpu.async_copy(xr_hbm.at[idx_d], b_buf, sb)

    def _wait(idx_s, idx_d, a_buf, b_buf, sa, sb):
        pltpu.make_async_copy(xl_hbm.at[idx_s], a_buf, sa).wait()
        pltpu.make_async_copy(xr_hbm.at[idx_d], b_buf, sb).wait()

    def _compute(idx_d, a_buf, b_buf, y_buf, sd, so, p):
        @pl.when(p > 0)
        def _():
            pltpu.make_async_copy(y_buf, acc.at[sd], so).wait()

        @plsc.parallel_loop(0, CHUNK)
        def _edge(e):
            den = jnp.zeros((16,), jnp.float32)
            for h in range(H):
                av = a_buf[e, pl.ds(h * C, C)]
                bv = b_buf[e, pl.ds(h * C, C)]
                z = av + bv
                zl = jnp.maximum(z, 0.2 * z)
                logit = jnp.sum(zl * attv[h])
                exb = jnp.exp(jnp.broadcast_to(logit, (16,)))
                y_buf[e, pl.ds(h * C, C)] = av * exb
                den = jnp.where(laneh[h], exb, den)
            y_buf[e, pl.ds(128, 16)] = den

        for i in range(CHUNK // 16):
            sd[pl.ds(i * 16, 16)] = idx_d[pl.ds(i * 16, 16)]
        pltpu.async_copy(y_buf, acc.at[sd], so, add=True)

    _fetch(0, *sets[0])

    @pl.loop(0, NCHUNK // 2)
    def _pair(p):
        k = 2 * p
        _wait(*sets[0])
        _fetch(k + 1, *sets[1])
        _compute(idx_d0, a0, b0, y0, sd0, so0, p)
        _wait(*sets[1])

        @pl.when(p + 1 < NCHUNK // 2)
        def _():
            _fetch(k + 2, *sets[0])

        _compute(idx_d1, a1, b1, y1, sd1, so1, p)

    pltpu.make_async_copy(y0, acc.at[sd0], so0).wait()
    pltpu.make_async_copy(y1, acc.at[sd1], so1).wait()
    plsc.subcore_barrier()
    pltpu.sync_copy(acc.at[pl.ds(s * ZR, ZR)], out_hbm.at[c, pl.ds(s * ZR, ZR)])


def _mm2_body(x_ref, wl_ref, wr_ref, ol_ref, or_ref):
    xb = x_ref[...]
    ol_ref[...] = jnp.dot(xb, wl_ref[...], preferred_element_type=jnp.float32)
    or_ref[...] = jnp.dot(xb, wr_ref[...], preferred_element_type=jnp.float32)


def _mm2(x, wl, wr):
    return pl.pallas_call(
        _mm2_body,
        grid=(GRID_N,),
        in_specs=[
            pl.BlockSpec((RBLK, D), lambda i: (i, 0)),
            pl.BlockSpec((D, HID), lambda i: (0, 0)),
            pl.BlockSpec((D, HID), lambda i: (0, 0)),
        ],
        out_specs=[
            pl.BlockSpec((RBLK, HID), lambda i: (i, 0)),
            pl.BlockSpec((RBLK, HID), lambda i: (i, 0)),
        ],
        out_shape=[jax.ShapeDtypeStruct((N, HID), jnp.float32)] * 2,
    )(x, wl, wr)


def _node_post(acc_ref, b_ref, g_ref, be_ref):
    p = acc_ref[0] + acc_ref[1]
    y = p[:, 0:128]
    den8 = p[:, 128:136]
    hh = lax.broadcasted_iota(jnp.int32, (H, HID), 0)
    cc = lax.broadcasted_iota(jnp.int32, (H, HID), 1) // C
    sel = (hh == cc).astype(jnp.float32)
    den = jnp.dot(den8, sel, preferred_element_type=jnp.float32)
    hcat = y / (den + _EPS_DEN) + b_ref[...]
    mu = jnp.mean(hcat, axis=1, keepdims=True)
    var = jnp.mean((hcat - mu) ** 2, axis=1, keepdims=True)
    hn = g_ref[...] * (hcat - mu) * lax.rsqrt(var + _EPS_LN) + be_ref[...]
    return hn * jax.nn.sigmoid(hn)


def _post_mm_body(acc_ref, b_ref, g_ref, be_ref, wl_ref, wr_ref, ol_ref, or_ref):
    hs = _node_post(acc_ref, b_ref, g_ref, be_ref)
    ol_ref[...] = jnp.dot(hs, wl_ref[...], preferred_element_type=jnp.float32)
    or_ref[...] = jnp.dot(hs, wr_ref[...], preferred_element_type=jnp.float32)


def _post_mm(acc, b, g, be, wl, wr):
    return pl.pallas_call(
        _post_mm_body,
        grid=(GRID_N,),
        in_specs=[
            pl.BlockSpec((NC, RBLK, ROW), lambda i: (0, i, 0)),
            pl.BlockSpec((1, HID), lambda i: (0, 0)),
            pl.BlockSpec((1, HID), lambda i: (0, 0)),
            pl.BlockSpec((1, HID), lambda i: (0, 0)),
            pl.BlockSpec((D, HID), lambda i: (0, 0)),
            pl.BlockSpec((D, HID), lambda i: (0, 0)),
        ],
        out_specs=[
            pl.BlockSpec((RBLK, HID), lambda i: (i, 0)),
            pl.BlockSpec((RBLK, HID), lambda i: (i, 0)),
        ],
        out_shape=[jax.ShapeDtypeStruct((N, HID), jnp.float32)] * 2,
    )(acc, b, g, be, wl, wr)


def _final_body(acc_ref, b_ref, g_ref, be_ref, batch_ref, o_ref, sums_ref, cnt_ref):
    i = pl.program_id(0)
    hs = _node_post(acc_ref, b_ref, g_ref, be_ref)
    bk = batch_ref[0, 0, :]
    onehot = (bk[:, None] == lax.broadcasted_iota(jnp.int32, (RBLK, NG), 1))
    onehot = onehot.astype(jnp.float32)
    dnums = (((0,), (0,)), ((), ()))

    @pl.when(i == 0)
    def _():
        sums_ref[...] = jnp.zeros_like(sums_ref)
        cnt_ref[...] = jnp.zeros_like(cnt_ref)

    sums_ref[...] += lax.dot_general(onehot, hs, dnums,
                                     preferred_element_type=jnp.float32)
    cnt_ref[...] += lax.dot_general(onehot, jnp.ones((RBLK, HID), jnp.float32),
                                    dnums, preferred_element_type=jnp.float32)

    @pl.when(i == GRID_N - 1)
    def _():
        o_ref[...] = sums_ref[...] / jnp.maximum(cnt_ref[...], 1.0)


def _final(acc, b, g, be, batch3):
    return pl.pallas_call(
        _final_body,
        grid=(GRID_N,),
        in_specs=[
            pl.BlockSpec((NC, RBLK, ROW), lambda i: (0, i, 0)),
            pl.BlockSpec((1, HID), lambda i: (0, 0)),
            pl.BlockSpec((1, HID), lambda i: (0, 0)),
            pl.BlockSpec((1, HID), lambda i: (0, 0)),
            pl.BlockSpec((1, 1, RBLK), lambda i: (i, 0, 0)),
        ],
        out_specs=pl.BlockSpec((NG, HID), lambda i: (0, 0)),
        out_shape=jax.ShapeDtypeStruct((NG, HID), jnp.float32),
        scratch_shapes=[
            pltpu.VMEM((NG, HID), jnp.float32),
            pltpu.VMEM((NG, HID), jnp.float32),
        ],
    )(acc, b, g, be, batch3)


def kernel(x, edge_index, batch, Wl0, Wr0, att0, b0, g0, be0,
           Wl1, Wr1, att1, b1, g1, be1):
    loopi = jnp.arange(N, dtype=jnp.int32)
    npad = EP - (E + N)
    src = jnp.concatenate(
        [edge_index[0].astype(jnp.int32), loopi, jnp.zeros((npad,), jnp.int32)])
    dst = jnp.concatenate(
        [edge_index[1].astype(jnp.int32), loopi, jnp.full((npad,), N, jnp.int32)])
    zrow = jnp.zeros((ZR, ROW), jnp.float32)
    zpad = jnp.zeros((1, HID), jnp.float32)
    b0r, g0r, be0r = b0[None, :], g0[None, :], be0[None, :]
    b1r, g1r, be1r = b1[None, :], g1[None, :], be1[None, :]
    batch3 = batch.astype(jnp.int32).reshape(GRID_N, 1, RBLK)

    xl0, xr0 = _mm2(x, Wl0, Wr0)
    xr0p = jnp.concatenate([xr0, zpad], axis=0)
    acc0 = _sc_edge(xl0, xr0p, src, dst, att0, zrow)
    xl1, xr1 = _post_mm(acc0, b0r, g0r, be0r, Wl1, Wr1)
    xr1p = jnp.concatenate([xr1, zpad], axis=0)
    acc1 = _sc_edge(xl1, xr1p, src, dst, att1, zrow)
    return _final(acc1, b1r, g1r, be1r, batch3)


# speedup vs baseline: 2.4562x; 1.4031x over previous
import dataclasses
import functools

import jax
import jax.numpy as jnp
from jax import lax
from jax.experimental import pallas as pl
from jax.experimental.pallas import tpu as pltpu
from jax.experimental.pallas import tpu_sc as plsc

N = 10000
D = 128
H = 8
C = 16
HID = 128
NG = 64
E = 320000

NC = 2
NS = 16
NW = NC * NS
EP = 331776
EPW = EP // NW
CHUNK = 48
NCHUNK = EPW // CHUNK
ROW = 144
ACC_R = 10112
ZR = ACC_R // NS

RBLK = 1000
GRID_N = N // RBLK

_EPS_DEN = 1e-16
_EPS_LN = 1e-5


def _compiler_params():
    cp = pltpu.CompilerParams()
    fields = pltpu.CompilerParams.__dataclass_fields__
    if "needs_layout_passes" in fields:
        cp = dataclasses.replace(cp, needs_layout_passes=False)
    if "use_tc_tiling_on_sc" in fields:
        cp = dataclasses.replace(cp, use_tc_tiling_on_sc=False)
    return cp


@functools.lru_cache(maxsize=1)
def _build_sc_edge():
    mesh = plsc.VectorSubcoreMesh(core_axis_name="c", subcore_axis_name="s")
    return pl.kernel(
        _sc_edge_body,
        out_type=jax.ShapeDtypeStruct((NC, ACC_R, ROW), jnp.float32),
        mesh=mesh,
        scratch_types=[
            pltpu.VMEM((CHUNK,), jnp.int32),
            pltpu.VMEM((CHUNK,), jnp.int32),
            pltpu.VMEM((CHUNK, D), jnp.float32),
            pltpu.VMEM((CHUNK, D), jnp.float32),
            pltpu.VMEM((CHUNK,), jnp.int32),
            pltpu.VMEM((CHUNK,), jnp.int32),
            pltpu.VMEM((CHUNK, D), jnp.float32),
            pltpu.VMEM((CHUNK, D), jnp.float32),
            pltpu.VMEM((CHUNK, ROW), jnp.float32),
            pltpu.VMEM((CHUNK, ROW), jnp.float32),
            pltpu.VMEM((CHUNK,), jnp.int32),
            pltpu.VMEM((CHUNK,), jnp.int32),
            pltpu.VMEM((H, C), jnp.float32),
            pltpu.VMEM_SHARED((ACC_R, ROW), jnp.float32),
            pltpu.SemaphoreType.DMA,
            pltpu.SemaphoreType.DMA,
            pltpu.SemaphoreType.DMA,
            pltpu.SemaphoreType.DMA,
            pltpu.SemaphoreType.DMA,
            pltpu.SemaphoreType.DMA,
            pltpu.SemaphoreType.DMA,
            pltpu.SemaphoreType.DMA,
            pltpu.SemaphoreType.DMA,
            pltpu.SemaphoreType.DMA,
        ],
        compiler_params=_compiler_params(),
    )


def _sc_edge(xl, xrp, src, dst, att, zrow):
    return _build_sc_edge()(xl, xrp, src, dst, att, zrow)


def _sc_edge_body(xl_hbm, xr_hbm, src_hbm, dst_hbm, att_hbm, zero_hbm, out_hbm,
                  idx_s0, idx_d0, a0, b0, idx_s1, idx_d1, a1, b1,
                  y0, y1, sd0, sd1, att_v, acc,
                  sa0, sb0, sa1, sb1, so0, so1,
                  si_s0, si_d0, si_s1, si_d1):
    c = lax.axis_index("c")
    s = lax.axis_index("s")
    wid = c * NS + s

    pltpu.sync_copy(att_hbm, att_v)
    pltpu.sync_copy(zero_hbm, acc.at[pl.ds(s * ZR, ZR)])
    plsc.subcore_barrier()

    attv = [att_v[h, :] for h in range(H)]
    lanes = lax.iota(jnp.int32, 16)
    laneh = [lanes == h for h in range(H)]
    sets = ((idx_s0, idx_d0, a0, b0, sa0, sb0, si_s0, si_d0),
            (idx_s1, idx_d1, a1, b1, sa1, sb1, si_s1, si_d1))

    def _fetch_idx(k, st):
        base = wid * EPW + k * CHUNK
        pltpu.async_copy(src_hbm.at[pl.ds(base, CHUNK)], st[0], st[6])
        pltpu.async_copy(dst_hbm.at[pl.ds(base, CHUNK)], st[1], st[7])

    def _gathers(k, st):
        base = wid * EPW + k * CHUNK
        pltpu.make_async_copy(src_hbm.at[pl.ds(base, CHUNK)], st[0], st[6]).wait()
        pltpu.make_async_copy(dst_hbm.at[pl.ds(base, CHUNK)], st[1], st[7]).wait()
        pltpu.async_copy(xl_hbm.at[st[0]], st[2], st[4])
        pltpu.async_copy(xr_hbm.at[st[1]], st[3], st[5])

    def _wait_rows(st):
        pltpu.make_async_copy(xl_hbm.at[st[0]], st[2], st[4]).wait()
        pltpu.make_async_copy(xr_hbm.at[st[1]], st[3], st[5]).wait()

    def _compute(idx_d, a_buf, b_buf, y_buf, sd, so, p, prefetch):
        @pl.when(p > 0)
        def _():
            pltpu.make_async_copy(y_buf, acc.at[sd], so).wait()
        for i in range(CHUNK // 16):
            sd[pl.ds(i * 16, 16)] = idx_d[pl.ds(i * 16, 16)]
        prefetch()

        @plsc.parallel_loop(0, CHUNK)
        def _edge(e):
            den = jnp.zeros((16,), jnp.float32)
            for h in range(H):
                av = a_buf[e, pl.ds(h * C, C)]
                bv = b_buf[e, pl.ds(h * C, C)]
                z = av + bv
                zl = jnp.maximum(z, 0.2 * z)
                logit = jnp.sum(zl * attv[h])
                exb = jnp.exp(jnp.broadcast_to(logit, (16,)))
                y_buf[e, pl.ds(h * C, C)] = av * exb
                den = jnp.where(laneh[h], exb, den)
            y_buf[e, pl.ds(128, 16)] = den

        pltpu.async_copy(y_buf, acc.at[sd], so, add=True)

    _fetch_idx(0, sets[0])
    _fetch_idx(1, sets[1])
    _gathers(0, sets[0])

    @pl.loop(0, NCHUNK // 2)
    def _pair(p):
        k = 2 * p
        more = p + 1 < NCHUNK // 2
        _wait_rows(sets[0])
        _gathers(k + 1, sets[1])

        def _pf0():
            @pl.when(more)
            def _():
                _fetch_idx(k + 2, sets[0])

        _compute(idx_d0, a0, b0, y0, sd0, so0, p, _pf0)
        _wait_rows(sets[1])

        @pl.when(more)
        def _():
            _gathers(k + 2, sets[0])

        def _pf1():
            @pl.when(more)
            def _():
                _fetch_idx(k + 3, sets[1])

        _compute(idx_d1, a1, b1, y1, sd1, so1, p, _pf1)

    pltpu.make_async_copy(y0, acc.at[sd0], so0).wait()
    pltpu.make_async_copy(y1, acc.at[sd1], so1).wait()
    plsc.subcore_barrier()
    pltpu.sync_copy(acc.at[pl.ds(s * ZR, ZR)], out_hbm.at[c, pl.ds(s * ZR, ZR)])


def _mm2_body(x_ref, wl_ref, wr_ref, ol_ref, or_ref):
    xb = x_ref[...]
    ol_ref[...] = jnp.dot(xb, wl_ref[...], preferred_element_type=jnp.float32)
    or_ref[...] = jnp.dot(xb, wr_ref[...], preferred_element_type=jnp.float32)


def _mm2(x, wl, wr):
    return pl.pallas_call(
        _mm2_body,
        grid=(GRID_N,),
        in_specs=[
            pl.BlockSpec((RBLK, D), lambda i: (i, 0)),
            pl.BlockSpec((D, HID), lambda i: (0, 0)),
            pl.BlockSpec((D, HID), lambda i: (0, 0)),
        ],
        out_specs=[
            pl.BlockSpec((RBLK, HID), lambda i: (i, 0)),
            pl.BlockSpec((RBLK, HID), lambda i: (i, 0)),
        ],
        out_shape=[jax.ShapeDtypeStruct((N, HID), jnp.float32)] * 2,
    )(x, wl, wr)


def _node_post(acc_ref, b_ref, g_ref, be_ref):
    p = acc_ref[0] + acc_ref[1]
    y = p[:, 0:128]
    den8 = p[:, 128:136]
    hh = lax.broadcasted_iota(jnp.int32, (H, HID), 0)
    cc = lax.broadcasted_iota(jnp.int32, (H, HID), 1) // C
    sel = (hh == cc).astype(jnp.float32)
    den = jnp.dot(den8, sel, preferred_element_type=jnp.float32)
    hcat = y / (den + _EPS_DEN) + b_ref[...]
    mu = jnp.mean(hcat, axis=1, keepdims=True)
    var = jnp.mean((hcat - mu) ** 2, axis=1, keepdims=True)
    hn = g_ref[...] * (hcat - mu) * lax.rsqrt(var + _EPS_LN) + be_ref[...]
    return hn * jax.nn.sigmoid(hn)


def _post_mm_body(acc_ref, b_ref, g_ref, be_ref, wl_ref, wr_ref, ol_ref, or_ref):
    hs = _node_post(acc_ref, b_ref, g_ref, be_ref)
    ol_ref[...] = jnp.dot(hs, wl_ref[...], preferred_element_type=jnp.float32)
    or_ref[...] = jnp.dot(hs, wr_ref[...], preferred_element_type=jnp.float32)


def _post_mm(acc, b, g, be, wl, wr):
    return pl.pallas_call(
        _post_mm_body,
        grid=(GRID_N,),
        in_specs=[
            pl.BlockSpec((NC, RBLK, ROW), lambda i: (0, i, 0)),
            pl.BlockSpec((1, HID), lambda i: (0, 0)),
            pl.BlockSpec((1, HID), lambda i: (0, 0)),
            pl.BlockSpec((1, HID), lambda i: (0, 0)),
            pl.BlockSpec((D, HID), lambda i: (0, 0)),
            pl.BlockSpec((D, HID), lambda i: (0, 0)),
        ],
        out_specs=[
            pl.BlockSpec((RBLK, HID), lambda i: (i, 0)),
            pl.BlockSpec((RBLK, HID), lambda i: (i, 0)),
        ],
        out_shape=[jax.ShapeDtypeStruct((N, HID), jnp.float32)] * 2,
    )(acc, b, g, be, wl, wr)


def _final_body(acc_ref, b_ref, g_ref, be_ref, batch_ref, o_ref, sums_ref, cnt_ref):
    i = pl.program_id(0)
    hs = _node_post(acc_ref, b_ref, g_ref, be_ref)
    bk = batch_ref[0, 0, :]
    onehot = (bk[:, None] == lax.broadcasted_iota(jnp.int32, (RBLK, NG), 1))
    onehot = onehot.astype(jnp.float32)
    dnums = (((0,), (0,)), ((), ()))

    @pl.when(i == 0)
    def _():
        sums_ref[...] = jnp.zeros_like(sums_ref)
        cnt_ref[...] = jnp.zeros_like(cnt_ref)

    sums_ref[...] += lax.dot_general(onehot, hs, dnums,
                                     preferred_element_type=jnp.float32)
    cnt_ref[...] += lax.dot_general(onehot, jnp.ones((RBLK, HID), jnp.float32),
                                    dnums, preferred_element_type=jnp.float32)

    @pl.when(i == GRID_N - 1)
    def _():
        o_ref[...] = sums_ref[...] / jnp.maximum(cnt_ref[...], 1.0)


def _final(acc, b, g, be, batch3):
    return pl.pallas_call(
        _final_body,
        grid=(GRID_N,),
        in_specs=[
            pl.BlockSpec((NC, RBLK, ROW), lambda i: (0, i, 0)),
            pl.BlockSpec((1, HID), lambda i: (0, 0)),
            pl.BlockSpec((1, HID), lambda i: (0, 0)),
            pl.BlockSpec((1, HID), lambda i: (0, 0)),
            pl.BlockSpec((1, 1, RBLK), lambda i: (i, 0, 0)),
        ],
        out_specs=pl.BlockSpec((NG, HID), lambda i: (0, 0)),
        out_shape=jax.ShapeDtypeStruct((NG, HID), jnp.float32),
        scratch_shapes=[
            pltpu.VMEM((NG, HID), jnp.float32),
            pltpu.VMEM((NG, HID), jnp.float32),
        ],
    )(acc, b, g, be, batch3)


def kernel(x, edge_index, batch, Wl0, Wr0, att0, b0, g0, be0,
           Wl1, Wr1, att1, b1, g1, be1):
    loopi = jnp.arange(N, dtype=jnp.int32)
    npad = EP - (E + N)
    src = jnp.concatenate(
        [edge_index[0].astype(jnp.int32), loopi, jnp.zeros((npad,), jnp.int32)])
    dst = jnp.concatenate(
        [edge_index[1].astype(jnp.int32), loopi, jnp.full((npad,), N, jnp.int32)])
    zrow = jnp.zeros((ZR, ROW), jnp.float32)
    zpad = jnp.zeros((1, HID), jnp.float32)
    b0r, g0r, be0r = b0[None, :], g0[None, :], be0[None, :]
    b1r, g1r, be1r = b1[None, :], g1[None, :], be1[None, :]
    batch3 = batch.astype(jnp.int32).reshape(GRID_N, 1, RBLK)

    xl0, xr0 = _mm2(x, Wl0, Wr0)
    xr0p = jnp.concatenate([xr0, zpad], axis=0)
    acc0 = _sc_edge(xl0, xr0p, src, dst, att0, zrow)
    xl1, xr1 = _post_mm(acc0, b0r, g0r, be0r, Wl1, Wr1)
    xr1p = jnp.concatenate([xr1, zpad], axis=0)
    acc1 = _sc_edge(xl1, xr1p, src, dst, att1, zrow)
    return _final(acc1, b1r, g1r, be1r, batch3)
